# baseline (speedup 1.0000x reference)
import functools

import jax
import jax.numpy as jnp
from jax.experimental import pallas as pl
from jax.experimental.pallas import tpu as pltpu

_NEG = -1e30


def _ln(x, g, b, eps=1e-5):
    m = jnp.mean(x, axis=-1, keepdims=True)
    d = x - m
    v = jnp.mean(d * d, axis=-1, keepdims=True)
    return d * jax.lax.rsqrt(v + eps) * g + b


def _dot(a, b):
    return jnp.dot(a, b, preferred_element_type=jnp.float32)


def _gnn_kernel(
    x_ref, adj_ref,
    ne_w1t_ref, ne_b1_ref, ne_g1_ref, ne_be1_ref,
    ne_w2t_ref, ne_b2_ref, ne_g2_ref, ne_be2_ref,
    wnode_ref, bnode_ref, mm_g1_ref, mm_be1_ref,
    mm_w2t_ref, mm_b2_ref,
    wct_ref, at_w2_ref, at_b2_ref,
    gru_wit_ref, gru_bi_ref,
    op_wt_ref, op_b_ref,
    out_ref,
):
    T = adj_ref.shape[0]
    H = mm_g1_ref.shape[-1]
    BB = x_ref.shape[0]
    E = x_ref.shape[-1]

    x = x_ref[...].reshape(BB * T, E)
    adj = adj_ref[...]
    mask = (adj > 0.0)[None, :, :]
    maskf = mask.astype(jnp.float32)

    h = _dot(x, ne_w1t_ref[...]) + ne_b1_ref[...]
    h = jnp.maximum(_ln(h, ne_g1_ref[...], ne_be1_ref[...]), 0.0)
    h = _dot(h, ne_w2t_ref[...]) + ne_b2_ref[...]
    h = jnp.maximum(_ln(h, ne_g2_ref[...], ne_be2_ref[...]), 0.0)

    mm_g1 = mm_g1_ref[...]
    mm_be1 = mm_be1_ref[...]
    at_w2 = at_w2_ref[...]
    at_b2 = at_b2_ref[0, 0]
    inv_h = 1.0 / H

    for _ in range(3):
        big = _dot(h, wnode_ref[...]) + bnode_ref[...]
        a = big[:, :H]
        b = big[:, H:2 * H]
        c = big[:, 2 * H:3 * H]
        gh = big[:, 3 * H:]

        da = a - jnp.mean(a, axis=-1, keepdims=True)
        db = b - jnp.mean(b, axis=-1, keepdims=True)
        na = jnp.sum(da * da, axis=-1, keepdims=True) * inv_h
        nb = jnp.sum(db * db, axis=-1, keepdims=True) * inv_h
        da3 = da.reshape(BB, T, H)
        db3 = db.reshape(BB, T, H)
        gram = jax.lax.dot_general(
            da3, db3, (((2,), (2,)), ((0,), (0,))),
            preferred_element_type=jnp.float32) * (2.0 * inv_h)
        v = (na.reshape(BB, T, 1) + gram + nb.reshape(BB, 1, T))
        r = jax.lax.rsqrt(v + 1e-5)

        dag = (da * mm_g1).reshape(BB, T, 1, H)
        dbg = (db * mm_g1).reshape(BB, 1, T, H)
        cj = c.reshape(BB, 1, T, H)

        NCH = 4
        CS = T // NCH
        us, ws = [], []
        for k in range(NCH):
            dag_k = dag[:, k * CS:(k + 1) * CS]
            r_k = r[:, k * CS:(k + 1) * CS, :, None]
            u_k = jnp.maximum((dag_k + dbg) * r_k + mm_be1, 0.0)
            u2_k = u_k.astype(jnp.bfloat16).reshape(BB * CS * T, H)
            t_k = _dot(u2_k, wct_ref[...]).reshape(BB, CS, T, H) + cj
            w_k = jnp.sum(jnp.tanh(t_k) * at_w2[None, None, :, :], axis=-1)
            us.append(u_k)
            ws.append(w_k)
        w = jnp.concatenate(ws, axis=1) + at_b2

        wl = jnp.where(mask, w, _NEG)
        p = jnp.exp(wl - jnp.max(wl, axis=1, keepdims=True))
        attn = p / jnp.sum(p, axis=1, keepdims=True) * maskf

        s = sum(
            jnp.sum(attn[:, k * CS:(k + 1) * CS, :, None] * us[k], axis=1)
            for k in range(NCH))
        colsum = jnp.sum(jnp.swapaxes(attn, 1, 2), axis=-1, keepdims=True)
        agg = (_dot(s.reshape(BB * T, H), mm_w2t_ref[...])
               + colsum.reshape(BB * T, 1) * mm_b2_ref[...])

        gi = _dot(agg, gru_wit_ref[...]) + gru_bi_ref[...]
        rg = jax.nn.sigmoid(gi[:, :H] + gh[:, :H])
        z = jax.nn.sigmoid(gi[:, H:2 * H] + gh[:, H:2 * H])
        n = jnp.tanh(gi[:, 2 * H:] + rg * gh[:, 2 * H:])
        h = (1.0 - z) * n + z * h

    out = _dot(h, op_wt_ref[...]) + op_b_ref[...]
    out_ref[...] = out.reshape(BB, T, E)


@jax.jit
def kernel(node_embeddings, adjacency_matrix,
           ne_w1, ne_b1, ne_g1, ne_be1, ne_w2, ne_b2, ne_g2, ne_be2,
           mm_w1, mm_b1, mm_g1, mm_be1, mm_w2, mm_b2,
           at_w1, at_b1, at_w2, at_b2,
           gru_wi, gru_bi, gru_wh, gru_bh,
           op_w, op_b):
    B, T, E = node_embeddings.shape
    H = ne_b1.shape[0]

    w1a = mm_w1[:, :H]
    w1b = mm_w1[:, H:]
    at_w1a = at_w1[:, :H]
    at_w1b = at_w1[:, H:]
    wc = at_w1b @ mm_w2
    att_bias = at_b1 + at_w1b @ mm_b2

    wnode = jnp.concatenate([w1a.T, w1b.T, at_w1a.T, gru_wh.T], axis=1)
    bnode = jnp.concatenate(
        [jnp.zeros_like(mm_b1), mm_b1, att_bias, gru_bh])[None, :]

    row = lambda v: v[None, :]
    args = (
        node_embeddings, adjacency_matrix,
        ne_w1.T, row(ne_b1), row(ne_g1), row(ne_be1),
        ne_w2.T, row(ne_b2), row(ne_g2), row(ne_be2),
        wnode, bnode, row(mm_g1), row(mm_be1),
        mm_w2.T, row(mm_b2),
        wc.T.astype(jnp.bfloat16), at_w2, at_b2[None, :],
        gru_wi.T, row(gru_bi),
        op_w.T, row(op_b),
    )

    BB = 16
    fixed = lambda shape: pl.BlockSpec(shape, lambda b: (0,) * len(shape))
    in_specs = [
        pl.BlockSpec((BB, T, E), lambda b: (b, 0, 0)),
        fixed((T, T)),
    ] + [fixed(a.shape) for a in args[2:]]

    return pl.pallas_call(
        _gnn_kernel,
        grid=(B // BB,),
        in_specs=in_specs,
        out_specs=pl.BlockSpec((BB, T, E), lambda b: (b, 0, 0)),
        out_shape=jax.ShapeDtypeStruct((B, T, E), jnp.float32),
        compiler_params=pltpu.CompilerParams(
            dimension_semantics=("arbitrary",),
        ),
    )(*args)


# speedup vs baseline: 1.0872x; 1.0872x over previous
import functools

import jax
import jax.numpy as jnp
from jax.experimental import pallas as pl
from jax.experimental.pallas import tpu as pltpu

_NEG = -1e30


def _ln(x, g, b, eps=1e-5):
    m = jnp.mean(x, axis=-1, keepdims=True)
    d = x - m
    v = jnp.mean(d * d, axis=-1, keepdims=True)
    return d * jax.lax.rsqrt(v + eps) * g + b


def _dot(a, b):
    return jnp.dot(a, b, preferred_element_type=jnp.float32)


def _gnn_kernel(
    x_ref, adj_ref,
    ne_w1t_ref, ne_b1_ref, ne_g1_ref, ne_be1_ref,
    ne_w2t_ref, ne_b2_ref, ne_g2_ref, ne_be2_ref,
    wnode_ref, bnode_ref, mm_g1_ref, mm_be1_ref,
    mm_w2t_ref, mm_b2_ref,
    wct_ref, at_w2_ref, at_b2_ref,
    gru_wit_ref, gru_bi_ref,
    op_wt_ref, op_b_ref,
    out_ref,
):
    T = adj_ref.shape[0]
    H = mm_g1_ref.shape[-1]
    BB = x_ref.shape[0]
    E = x_ref.shape[-1]

    x = x_ref[...].reshape(BB * T, E)
    adj = adj_ref[...]
    mask = (adj > 0.0)[None, :, :]
    maskf = mask.astype(jnp.float32)

    h = _dot(x, ne_w1t_ref[...]) + ne_b1_ref[...]
    h = jnp.maximum(_ln(h, ne_g1_ref[...], ne_be1_ref[...]), 0.0)
    h = _dot(h, ne_w2t_ref[...]) + ne_b2_ref[...]
    h = jnp.maximum(_ln(h, ne_g2_ref[...], ne_be2_ref[...]), 0.0)

    mm_g1 = mm_g1_ref[...]
    mm_be1 = mm_be1_ref[...]
    at_w2 = at_w2_ref[...]
    at_b2 = at_b2_ref[0, 0]
    inv_h = 1.0 / H

    for _ in range(3):
        big = _dot(h, wnode_ref[...]) + bnode_ref[...]
        a = big[:, :H]
        b = big[:, H:2 * H]
        c = big[:, 2 * H:3 * H]
        gh = big[:, 3 * H:]

        da = a - jnp.mean(a, axis=-1, keepdims=True)
        db = b - jnp.mean(b, axis=-1, keepdims=True)
        na = jnp.sum(da * da, axis=-1, keepdims=True) * inv_h
        nb = jnp.sum(db * db, axis=-1, keepdims=True) * inv_h
        da3 = da.reshape(BB, T, H)
        db3 = db.reshape(BB, T, H)
        gram = jax.lax.dot_general(
            da3, db3, (((2,), (2,)), ((0,), (0,))),
            preferred_element_type=jnp.float32) * (2.0 * inv_h)
        v = (na.reshape(BB, T, 1) + gram + nb.reshape(BB, 1, T))
        r = jax.lax.rsqrt(v + 1e-5)

        dag = (da * mm_g1).reshape(BB, T, 1, H)
        dbg = (db * mm_g1).reshape(BB, 1, T, H)
        u = jnp.maximum(
            (dag + dbg) * r[:, :, :, None] + mm_be1,
            0.0).astype(jnp.bfloat16)

        u2 = u.reshape(BB * T * T, H)
        t = (_dot(u2, wct_ref[...]).reshape(BB, T, T, H)
             + c.reshape(BB, 1, T, H))
        w = jnp.sum(jnp.tanh(t) * at_w2[None, None, :, :], axis=-1) + at_b2

        wl = jnp.where(mask, w, _NEG)
        p = jnp.exp(wl - jnp.max(wl, axis=1, keepdims=True))
        attn = p / jnp.sum(p, axis=1, keepdims=True) * maskf

        s = jnp.sum(attn[:, :, :, None] * u.astype(jnp.float32), axis=1)
        colsum = jnp.sum(jnp.swapaxes(attn, 1, 2), axis=-1, keepdims=True)
        agg = (_dot(s.reshape(BB * T, H), mm_w2t_ref[...])
               + colsum.reshape(BB * T, 1) * mm_b2_ref[...])

        gi = _dot(agg, gru_wit_ref[...]) + gru_bi_ref[...]
        rg = jax.nn.sigmoid(gi[:, :H] + gh[:, :H])
        z = jax.nn.sigmoid(gi[:, H:2 * H] + gh[:, H:2 * H])
        n = jnp.tanh(gi[:, 2 * H:] + rg * gh[:, 2 * H:])
        h = (1.0 - z) * n + z * h

    out = _dot(h, op_wt_ref[...]) + op_b_ref[...]
    out_ref[...] = out.reshape(BB, T, E)


@jax.jit
def kernel(node_embeddings, adjacency_matrix,
           ne_w1, ne_b1, ne_g1, ne_be1, ne_w2, ne_b2, ne_g2, ne_be2,
           mm_w1, mm_b1, mm_g1, mm_be1, mm_w2, mm_b2,
           at_w1, at_b1, at_w2, at_b2,
           gru_wi, gru_bi, gru_wh, gru_bh,
           op_w, op_b):
    B, T, E = node_embeddings.shape
    H = ne_b1.shape[0]

    w1a = mm_w1[:, :H]
    w1b = mm_w1[:, H:]
    at_w1a = at_w1[:, :H]
    at_w1b = at_w1[:, H:]
    wc = at_w1b @ mm_w2
    att_bias = at_b1 + at_w1b @ mm_b2

    wnode = jnp.concatenate([w1a.T, w1b.T, at_w1a.T, gru_wh.T], axis=1)
    bnode = jnp.concatenate(
        [jnp.zeros_like(mm_b1), mm_b1, att_bias, gru_bh])[None, :]

    row = lambda v: v[None, :]
    args = (
        node_embeddings, adjacency_matrix,
        ne_w1.T, row(ne_b1), row(ne_g1), row(ne_be1),
        ne_w2.T, row(ne_b2), row(ne_g2), row(ne_be2),
        wnode, bnode, row(mm_g1), row(mm_be1),
        mm_w2.T, row(mm_b2),
        wc.T.astype(jnp.bfloat16), at_w2, at_b2[None, :],
        gru_wi.T, row(gru_bi),
        op_w.T, row(op_b),
    )

    BB = 16
    fixed = lambda shape: pl.BlockSpec(shape, lambda b: (0,) * len(shape))
    in_specs = [
        pl.BlockSpec((BB, T, E), lambda b: (b, 0, 0)),
        fixed((T, T)),
    ] + [fixed(a.shape) for a in args[2:]]

    return pl.pallas_call(
        _gnn_kernel,
        grid=(B // BB,),
        in_specs=in_specs,
        out_specs=pl.BlockSpec((BB, T, E), lambda b: (b, 0, 0)),
        out_shape=jax.ShapeDtypeStruct((B, T, E), jnp.float32),
        compiler_params=pltpu.CompilerParams(
            dimension_semantics=("arbitrary",),
        ),
    )(*args)


# speedup vs baseline: 1.3284x; 1.2218x over previous
import jax
import jax.numpy as jnp
from jax.experimental import pallas as pl
from jax.experimental.pallas import tpu as pltpu

_NEG = -1e30


def _ln(x, g, b, eps=1e-5):
    m = jnp.mean(x, axis=-1, keepdims=True)
    d = x - m
    v = jnp.mean(d * d, axis=-1, keepdims=True)
    return d * jax.lax.rsqrt(v + eps) * g + b


def _dot_t(a, w):
    return jax.lax.dot_general(
        a, w, (((1,), (1,)), ((), ())), preferred_element_type=jnp.float32)


def _gnn_kernel(
    x_ref, adj_ref,
    ne_w1_ref, ne_b1_ref, ne_g1_ref, ne_be1_ref,
    ne_w2_ref, ne_b2_ref, ne_g2_ref, ne_be2_ref,
    mm_w1_ref, mm_b1_ref, mm_g1_ref, mm_be1_ref,
    mm_w2_ref, mm_b2_ref,
    at_w1_ref, at_b1_ref, at_w2_ref, at_b2_ref,
    gru_wi_ref, gru_bi_ref, gru_wh_ref, gru_bh_ref,
    op_w_ref, op_b_ref,
    out_ref,
):
    T = adj_ref.shape[0]
    H = mm_g1_ref.shape[-1]
    BB = x_ref.shape[0]
    E = x_ref.shape[-1]

    x = x_ref[...].reshape(BB * T, E)
    adj = adj_ref[...]
    mask = (adj > 0.0)[None, :, :]
    maskf = mask.astype(jnp.float32)

    mm_w2 = mm_w2_ref[...]
    at_w1b = at_w1_ref[:, H:]
    wct = jax.lax.dot_general(
        mm_w2, at_w1b, (((0,), (1,)), ((), ())),
        preferred_element_type=jnp.float32).astype(jnp.bfloat16)
    att_bias = at_b1_ref[...] + _dot_t(mm_b2_ref[...], at_w1b)
    wcat = jnp.concatenate(
        [mm_w1_ref[:, :H], mm_w1_ref[:, H:], at_w1_ref[:, :H],
         gru_wh_ref[...]], axis=0)
    bcat = jnp.concatenate(
        [jnp.zeros_like(mm_b1_ref[...]), mm_b1_ref[...], att_bias,
         gru_bh_ref[...]], axis=1)

    h = _dot_t(x, ne_w1_ref[...]) + ne_b1_ref[...]
    h = jnp.maximum(_ln(h, ne_g1_ref[...], ne_be1_ref[...]), 0.0)
    h = _dot_t(h, ne_w2_ref[...]) + ne_b2_ref[...]
    h = jnp.maximum(_ln(h, ne_g2_ref[...], ne_be2_ref[...]), 0.0)

    mm_g1 = mm_g1_ref[...]
    mm_be1 = mm_be1_ref[...]
    at_w2 = at_w2_ref[...]
    at_b2 = at_b2_ref[0, 0]
    inv_h = 1.0 / H

    for _ in range(3):
        big = _dot_t(h, wcat) + bcat
        a = big[:, :H]
        b = big[:, H:2 * H]
        c = big[:, 2 * H:3 * H]
        gh = big[:, 3 * H:]

        da = a - jnp.mean(a, axis=-1, keepdims=True)
        db = b - jnp.mean(b, axis=-1, keepdims=True)
        na = jnp.sum(da * da, axis=-1, keepdims=True) * inv_h
        nb = jnp.sum(db * db, axis=-1, keepdims=True) * inv_h
        gram = jax.lax.dot_general(
            da.reshape(BB, T, H), db.reshape(BB, T, H),
            (((2,), (2,)), ((0,), (0,))),
            preferred_element_type=jnp.float32) * (2.0 * inv_h)
        v = na.reshape(BB, T, 1) + gram + nb.reshape(BB, 1, T)
        r = jax.lax.rsqrt(v + 1e-5)

        dag = (da * mm_g1).reshape(BB, T, 1, H)
        dbg = (db * mm_g1).reshape(BB, 1, T, H)
        u = jnp.maximum(
            (dag + dbg) * r[:, :, :, None] + mm_be1,
            0.0).astype(jnp.bfloat16)

        t = (jnp.dot(u.reshape(BB * T * T, H), wct,
                     preferred_element_type=jnp.float32).reshape(BB, T, T, H)
             + c.reshape(BB, 1, T, H))
        w = jnp.sum(jnp.tanh(t) * at_w2[None, None, :, :], axis=-1) + at_b2

        wl = jnp.where(mask, w, _NEG)
        p = jnp.exp(wl - jnp.max(wl, axis=1, keepdims=True))
        attn = p / jnp.sum(p, axis=1, keepdims=True) * maskf

        s = jnp.sum(attn[:, :, :, None] * u.astype(jnp.float32), axis=1)
        colsum = jnp.sum(jnp.swapaxes(attn, 1, 2), axis=-1, keepdims=True)
        agg = (_dot_t(s.reshape(BB * T, H), mm_w2)
               + colsum.reshape(BB * T, 1) * mm_b2_ref[...])

        gi = _dot_t(agg, gru_wi_ref[...]) + gru_bi_ref[...]
        rg = jax.nn.sigmoid(gi[:, :H] + gh[:, :H])
        z = jax.nn.sigmoid(gi[:, H:2 * H] + gh[:, H:2 * H])
        n = jnp.tanh(gi[:, 2 * H:] + rg * gh[:, 2 * H:])
        h = (1.0 - z) * n + z * h

    out = _dot_t(h, op_w_ref[...]) + op_b_ref[...]
    out_ref[...] = out.reshape(BB, T, E)


@jax.jit
def kernel(node_embeddings, adjacency_matrix,
           ne_w1, ne_b1, ne_g1, ne_be1, ne_w2, ne_b2, ne_g2, ne_be2,
           mm_w1, mm_b1, mm_g1, mm_be1, mm_w2, mm_b2,
           at_w1, at_b1, at_w2, at_b2,
           gru_wi, gru_bi, gru_wh, gru_bh,
           op_w, op_b):
    B, T, E = node_embeddings.shape
    H = ne_b1.shape[0]

    row = lambda v: v[None, :]
    args = (
        node_embeddings, adjacency_matrix,
        ne_w1, row(ne_b1), row(ne_g1), row(ne_be1),
        ne_w2, row(ne_b2), row(ne_g2), row(ne_be2),
        mm_w1, row(mm_b1), row(mm_g1), row(mm_be1),
        mm_w2, row(mm_b2),
        at_w1, row(at_b1), at_w2, at_b2[None, :],
        gru_wi, row(gru_bi), gru_wh, row(gru_bh),
        op_w, row(op_b),
    )

    BB = 16
    fixed = lambda shape: pl.BlockSpec(shape, lambda bq: (0,) * len(shape))
    in_specs = [
        pl.BlockSpec((BB, T, E), lambda bq: (bq, 0, 0)),
        fixed((T, T)),
    ] + [fixed(a.shape) for a in args[2:]]

    return pl.pallas_call(
        _gnn_kernel,
        grid=(B // BB,),
        in_specs=in_specs,
        out_specs=pl.BlockSpec((BB, T, E), lambda bq: (bq, 0, 0)),
        out_shape=jax.ShapeDtypeStruct((B, T, E), jnp.float32),
        compiler_params=pltpu.CompilerParams(
            dimension_semantics=("arbitrary",),
        ),
    )(*args)


# speedup vs baseline: 1.3843x; 1.0420x over previous
import jax
import jax.numpy as jnp
from jax.experimental import pallas as pl
from jax.experimental.pallas import tpu as pltpu

_NEG = -1e30


def _ln(x, g, b, eps=1e-5):
    m = jnp.mean(x, axis=-1, keepdims=True)
    d = x - m
    v = jnp.mean(d * d, axis=-1, keepdims=True)
    return d * jax.lax.rsqrt(v + eps) * g + b


def _dot_t(a, w):
    return jax.lax.dot_general(
        a, w, (((1,), (1,)), ((), ())), preferred_element_type=jnp.float32)


def _gnn_kernel(
    x_ref, adj_ref,
    ne_w1_ref, ne_b1_ref, ne_g1_ref, ne_be1_ref,
    ne_w2_ref, ne_b2_ref, ne_g2_ref, ne_be2_ref,
    mm_w1_ref, mm_b1_ref, mm_g1_ref, mm_be1_ref,
    mm_w2_ref, mm_b2_ref,
    at_w1_ref, at_b1_ref, at_w2_ref, at_b2_ref,
    gru_wi_ref, gru_bi_ref, gru_wh_ref, gru_bh_ref,
    op_w_ref, op_b_ref,
    out_ref,
):
    T = adj_ref.shape[0]
    H = mm_g1_ref.shape[-1]
    BB = x_ref.shape[0]
    E = x_ref.shape[-1]

    x = x_ref[...].reshape(BB * T, E)
    adj = adj_ref[...]
    mask = (adj > 0.0)[None, :, :]
    maskf = mask.astype(jnp.float32)

    mm_w2 = mm_w2_ref[...]
    at_w1b = at_w1_ref[:, H:]
    wct = jax.lax.dot_general(
        mm_w2, at_w1b, (((0,), (1,)), ((), ())),
        preferred_element_type=jnp.float32).astype(jnp.bfloat16)
    att_bias = at_b1_ref[...] + _dot_t(mm_b2_ref[...], at_w1b)
    wcat = jnp.concatenate(
        [mm_w1_ref[:, :H], mm_w1_ref[:, H:], at_w1_ref[:, :H],
         gru_wh_ref[...]], axis=0)
    bcat = jnp.concatenate(
        [jnp.zeros_like(mm_b1_ref[...]), mm_b1_ref[...], att_bias,
         gru_bh_ref[...]], axis=1)

    h = _dot_t(x, ne_w1_ref[...]) + ne_b1_ref[...]
    h = jnp.maximum(_ln(h, ne_g1_ref[...], ne_be1_ref[...]), 0.0)
    h = _dot_t(h, ne_w2_ref[...]) + ne_b2_ref[...]
    h = jnp.maximum(_ln(h, ne_g2_ref[...], ne_be2_ref[...]), 0.0)

    mm_g1 = mm_g1_ref[...]
    mm_be1 = mm_be1_ref[...]
    at_w2 = at_w2_ref[...]
    at_b2 = at_b2_ref[0, 0]
    inv_h = 1.0 / H

    for _ in range(3):
        big = _dot_t(h, wcat) + bcat
        a = big[:, :H]
        b = big[:, H:2 * H]
        c = big[:, 2 * H:3 * H]
        gh = big[:, 3 * H:]

        da = a - jnp.mean(a, axis=-1, keepdims=True)
        db = b - jnp.mean(b, axis=-1, keepdims=True)
        na = jnp.sum(da * da, axis=-1, keepdims=True) * inv_h
        nb = jnp.sum(db * db, axis=-1, keepdims=True) * inv_h
        gram = jax.lax.dot_general(
            da.reshape(BB, T, H), db.reshape(BB, T, H),
            (((2,), (2,)), ((0,), (0,))),
            preferred_element_type=jnp.float32) * (2.0 * inv_h)
        v = na.reshape(BB, T, 1) + gram + nb.reshape(BB, 1, T)
        r = jax.lax.rsqrt(v + 1e-5)

        dag = (da * mm_g1).astype(jnp.bfloat16).reshape(BB, T, 1, H)
        dbg = (db * mm_g1).astype(jnp.bfloat16).reshape(BB, 1, T, H)
        rb = r.astype(jnp.bfloat16)[:, :, :, None]
        u = jnp.maximum((dag + dbg) * rb + mm_be1.astype(jnp.bfloat16),
                        jnp.bfloat16(0.0))

        t = (jnp.dot(u.reshape(BB * T * T, H), wct,
                     preferred_element_type=jnp.float32).reshape(BB, T, T, H)
             + c.reshape(BB, 1, T, H))
        w = jnp.sum(jnp.tanh(t) * at_w2[None, None, :, :], axis=-1) + at_b2

        wl = jnp.where(mask, w, _NEG)
        p = jnp.exp(wl - jnp.max(wl, axis=1, keepdims=True))
        attn = p / jnp.sum(p, axis=1, keepdims=True) * maskf

        s = jnp.sum(attn[:, :, :, None] * u.astype(jnp.float32), axis=1)
        colsum = jnp.sum(jnp.swapaxes(attn, 1, 2), axis=-1, keepdims=True)
        agg = (_dot_t(s.reshape(BB * T, H), mm_w2)
               + colsum.reshape(BB * T, 1) * mm_b2_ref[...])

        gi = _dot_t(agg, gru_wi_ref[...]) + gru_bi_ref[...]
        rg = jax.nn.sigmoid(gi[:, :H] + gh[:, :H])
        z = jax.nn.sigmoid(gi[:, H:2 * H] + gh[:, H:2 * H])
        n = jnp.tanh(gi[:, 2 * H:] + rg * gh[:, 2 * H:])
        h = (1.0 - z) * n + z * h

    out = _dot_t(h, op_w_ref[...]) + op_b_ref[...]
    out_ref[...] = out.reshape(BB, T, E)


@jax.jit
def kernel(node_embeddings, adjacency_matrix,
           ne_w1, ne_b1, ne_g1, ne_be1, ne_w2, ne_b2, ne_g2, ne_be2,
           mm_w1, mm_b1, mm_g1, mm_be1, mm_w2, mm_b2,
           at_w1, at_b1, at_w2, at_b2,
           gru_wi, gru_bi, gru_wh, gru_bh,
           op_w, op_b):
    B, T, E = node_embeddings.shape
    H = ne_b1.shape[0]

    row = lambda v: v[None, :]
    args = (
        node_embeddings, adjacency_matrix,
        ne_w1, row(ne_b1), row(ne_g1), row(ne_be1),
        ne_w2, row(ne_b2), row(ne_g2), row(ne_be2),
        mm_w1, row(mm_b1), row(mm_g1), row(mm_be1),
        mm_w2, row(mm_b2),
        at_w1, row(at_b1), at_w2, at_b2[None, :],
        gru_wi, row(gru_bi), gru_wh, row(gru_bh),
        op_w, row(op_b),
    )

    BB = 16
    fixed = lambda shape: pl.BlockSpec(shape, lambda bq: (0,) * len(shape))
    in_specs = [
        pl.BlockSpec((BB, T, E), lambda bq: (bq, 0, 0)),
        fixed((T, T)),
    ] + [fixed(a.shape) for a in args[2:]]

    return pl.pallas_call(
        _gnn_kernel,
        grid=(B // BB,),
        in_specs=in_specs,
        out_specs=pl.BlockSpec((BB, T, E), lambda bq: (bq, 0, 0)),
        out_shape=jax.ShapeDtypeStruct((B, T, E), jnp.float32),
        compiler_params=pltpu.CompilerParams(
            dimension_semantics=("arbitrary",),
        ),
    )(*args)
